# baseline (speedup 1.0000x reference)
import functools

import jax
import jax.numpy as jnp
from jax import lax
from jax.experimental import pallas as pl
from jax.experimental.pallas import tpu as pltpu
from jax.experimental.pallas import tpu_sc as plsc

NC = 2
NS = 16
NW = NC * NS
CHUNK = 128
LANES = 16


def kernel(QR, symbols):
    B, L = QR.shape
    V, D = symbols.shape
    N = B * L
    assert N % (NW * CHUNK) == 0 and CHUNK % LANES == 0
    n_chunks = N // (NW * CHUNK)
    n_rg = CHUNK // LANES
    idx = (QR.reshape(n_chunks, NW, CHUNK).transpose(1, 0, 2)
           .reshape(NW, n_chunks * CHUNK))
    table_flat = symbols.reshape(V * D)

    mesh = plsc.VectorSubcoreMesh(core_axis_name="c", subcore_axis_name="s")

    @functools.partial(
        pl.kernel,
        mesh=mesh,
        compiler_params=pltpu.CompilerParams(needs_layout_passes=False),
        out_type=jax.ShapeDtypeStruct((N * D,), jnp.float32),
        scratch_types=[
            pltpu.VMEM((V * D,), jnp.float32),
            pltpu.VMEM((n_chunks * CHUNK,), jnp.int32),
            pltpu.VMEM((CHUNK * D,), jnp.float32),
            pltpu.VMEM((CHUNK * D,), jnp.float32),
            pltpu.SemaphoreType.DMA,
            pltpu.SemaphoreType.DMA,
        ],
    )
    def gather_kernel(table_hbm, idx_hbm, out_hbm, table_v, idxg_v,
                      obuf0, obuf1, ss0, ss1):
        wid = lax.axis_index("s") * NC + lax.axis_index("c")
        pltpu.sync_copy(table_hbm, table_v)
        pltpu.sync_copy(idx_hbm.at[wid], idxg_v)
        obufs, ssems = (obuf0, obuf1), (ss0, ss1)
        iota = lax.iota(jnp.int32, LANES)
        rowbase0 = iota * D

        def s_copy(c, b):
            return pltpu.make_async_copy(
                obufs[b],
                out_hbm.at[pl.ds((c * NW + wid) * (CHUNK * D), CHUNK * D)],
                ssems[b])

        def fill(c, b):
            def rg_body(rg, carry):
                idx16 = idxg_v[pl.ds((c * n_rg + rg) * LANES, LANES)]
                src0 = idx16 * D
                dst0 = rowbase0 + rg * (LANES * D)
                for col in range(D):
                    vals = plsc.load_gather(table_v, [src0 + col])
                    plsc.store_scatter(obufs[b], [dst0 + col], vals)
                return carry

            lax.fori_loop(0, n_rg, rg_body, 0)

        for c in range(2):
            fill(c, c)
            s_copy(c, c).start()

        def body(g, carry):
            for b in range(2):
                c = g * 2 + b
                s_copy(c - 2, b).wait()
                fill(c, b)
                s_copy(c, b).start()
            return carry

        lax.fori_loop(1, n_chunks // 2, body, 0)
        for b in range(2):
            s_copy(n_chunks - 2 + b, b).wait()

    out = gather_kernel(table_flat, idx)
    return out.reshape(B, L, D)


# speedup vs baseline: 2.9942x; 2.9942x over previous
import functools

import jax
import jax.numpy as jnp
from jax import lax
from jax.experimental import pallas as pl
from jax.experimental.pallas import tpu as pltpu
from jax.experimental.pallas import tpu_sc as plsc

NC = 2
NS = 16
NW = NC * NS
CHUNK = 128
LANES = 16


def kernel(QR, symbols):
    B, L = QR.shape
    V, D = symbols.shape
    N = B * L
    assert N % (NW * CHUNK) == 0 and CHUNK % LANES == 0
    n_chunks = N // (NW * CHUNK)
    n_rg = CHUNK // LANES
    idx = (QR.reshape(n_chunks, NW, CHUNK).transpose(1, 0, 2)
           .reshape(NW, n_chunks * CHUNK))
    table_flat = symbols.reshape(V * D)

    mesh = plsc.VectorSubcoreMesh(core_axis_name="c", subcore_axis_name="s")

    @functools.partial(
        pl.kernel,
        mesh=mesh,
        compiler_params=pltpu.CompilerParams(needs_layout_passes=False),
        out_type=jax.ShapeDtypeStruct((N * D,), jnp.float32),
        scratch_types=[
            pltpu.VMEM((V * D,), jnp.float32),
            pltpu.VMEM((n_chunks * CHUNK,), jnp.int32),
            pltpu.VMEM((CHUNK * D,), jnp.float32),
            pltpu.VMEM((CHUNK * D,), jnp.float32),
            pltpu.SemaphoreType.DMA,
            pltpu.SemaphoreType.DMA,
        ],
    )
    def gather_kernel(table_hbm, idx_hbm, out_hbm, table_v, idxg_v,
                      obuf0, obuf1, ss0, ss1):
        wid = lax.axis_index("s") * NC + lax.axis_index("c")
        pltpu.sync_copy(table_hbm, table_v)
        pltpu.sync_copy(idx_hbm.at[wid], idxg_v)
        obufs, ssems = (obuf0, obuf1), (ss0, ss1)

        def s_copy(c, b):
            return pltpu.make_async_copy(
                obufs[b],
                out_hbm.at[pl.ds((c * NW + wid) * (CHUNK * D), CHUNK * D)],
                ssems[b])

        def fill(c, b):
            def rg_body(rg, carry):
                idx16 = idxg_v[pl.ds((c * n_rg + rg) * LANES, LANES)]
                for r in range(LANES):
                    base = idx16[r] * D
                    row = (rg * LANES + r) * D
                    for j in range(0, D, LANES):
                        obufs[b][pl.ds(row + j, LANES)] = (
                            table_v[pl.ds(base + j, LANES)])
                return carry

            lax.fori_loop(0, n_rg, rg_body, 0)

        for c in range(2):
            fill(c, c)
            s_copy(c, c).start()

        def body(g, carry):
            for b in range(2):
                c = g * 2 + b
                s_copy(c - 2, b).wait()
                fill(c, b)
                s_copy(c, b).start()
            return carry

        lax.fori_loop(1, n_chunks // 2, body, 0)
        for b in range(2):
            s_copy(n_chunks - 2 + b, b).wait()

    out = gather_kernel(table_flat, idx)
    return out.reshape(B, L, D)


# speedup vs baseline: 4.9287x; 1.6461x over previous
import functools

import jax
import jax.numpy as jnp
from jax import lax
from jax.experimental import pallas as pl
from jax.experimental.pallas import tpu as pltpu
from jax.experimental.pallas import tpu_sc as plsc

NC = 2
NS = 16
NW = NC * NS
CHUNK = 128
LANES = 16


def kernel(QR, symbols):
    B, L = QR.shape
    V, D = symbols.shape
    N = B * L
    assert N % (NW * CHUNK) == 0 and CHUNK % LANES == 0
    n_chunks = N // (NW * CHUNK)
    n_rg = CHUNK // LANES
    idx = (QR.reshape(n_chunks, NW, CHUNK).transpose(1, 0, 2)
           .reshape(NW, n_chunks * CHUNK))
    table_flat = symbols.reshape(V * D)

    mesh = plsc.VectorSubcoreMesh(core_axis_name="c", subcore_axis_name="s")

    @functools.partial(
        pl.kernel,
        mesh=mesh,
        compiler_params=pltpu.CompilerParams(needs_layout_passes=False),
        out_type=jax.ShapeDtypeStruct((N * D,), jnp.float32),
        scratch_types=[
            pltpu.VMEM((V * D,), jnp.float32),
            pltpu.VMEM((n_chunks * CHUNK,), jnp.int32),
            pltpu.VMEM((CHUNK * D,), jnp.float32),
            pltpu.SemaphoreType.DMA,
            pltpu.SemaphoreType.DMA,
        ],
    )
    def gather_kernel(table_hbm, idx_hbm, out_hbm, table_v, idxg_v,
                      dummy_v, ss0, ss1):
        wid = lax.axis_index("s") * NC + lax.axis_index("c")
        pltpu.sync_copy(table_hbm, table_v)
        pltpu.sync_copy(idx_hbm.at[wid], idxg_v)
        ssems = (ss0, ss1)

        def fire(c, b):
            def rg_body(rg, carry):
                idx16 = idxg_v[pl.ds((c * n_rg + rg) * LANES, LANES)]
                out0 = ((c * NW + wid) * CHUNK + rg * LANES) * D
                for r in range(LANES):
                    base = idx16[r] * D
                    pltpu.make_async_copy(
                        table_v.at[pl.ds(base, D)],
                        out_hbm.at[pl.ds(out0 + r * D, D)],
                        ssems[b]).start()
                return carry

            lax.fori_loop(0, n_rg, rg_body, 0)

        def drain(b):
            pltpu.make_async_copy(
                out_hbm.at[pl.ds(0, CHUNK * D)], dummy_v, ssems[b]).wait()

        for c in range(2):
            fire(c, c)

        def body(g, carry):
            for b in range(2):
                c = g * 2 + b
                drain(b)
                fire(c, b)
            return carry

        lax.fori_loop(1, n_chunks // 2, body, 0)
        for b in range(2):
            drain(b)

    out = gather_kernel(table_flat, idx)
    return out.reshape(B, L, D)
